# baseline (speedup 1.0000x reference)
import numpy as np
import jax, jax.numpy as jnp
from jax.experimental import pallas as pl

RADIAL_ETA = 16.0
ANGULAR_ETA = 8.0
RADIAL_DIST_DIVISIONS = 16
ANGULAR_DIST_DIVISIONS = 4
ZETA = 32.0
ANGLE_SECTIONS = 4
RADIAL_START = 0.8
ANGULAR_START = 0.8
CUTOFF = 5.2
ANGULAR_CUTOFF = 3.5
N_SPECIES = 10

_VAL = np.array([
    [0.0, 0.0, 0.0, 0.0],
    [0.5, 0.0, 0.0, 0.0],
    [1.0, 0.0, 0.0, 0.0],
    [0.5, 0.0, 0.0, 0.0],
    [1.0, 0.0, 0.0, 0.0],
    [1.0, 1.0 / 6.0, 0.0, 0.0],
    [1.0, 2.0 / 6.0, 0.0, 0.0],
    [1.0, 3.0 / 6.0, 0.0, 0.0],
    [1.0, 4.0 / 6.0, 0.0, 0.0],
    [1.0, 5.0 / 6.0, 0.0, 0.0],
], dtype=np.float32)


def _identity(x_ref, o_ref):
    o_ref[...] = x_ref[...]


def kernel(species, distances, switch, edge_src, edge_dst, angles,
           ang_distances, ang_switch, ang_edge_dst, central_atom,
           angle_src, angle_dst):
    n_atoms = species.shape[0]
    d2 = distances.reshape(6250, 128)
    d2 = pl.pallas_call(
        _identity,
        out_shape=jax.ShapeDtypeStruct(d2.shape, d2.dtype),
    )(d2)
    distances = d2.reshape(-1)

    valence = jnp.asarray(_VAL)[species]

    shiftR = jnp.asarray(
        np.linspace(RADIAL_START, CUTOFF, RADIAL_DIST_DIVISIONS + 1)[None, :-1],
        dtype=distances.dtype)
    x2 = RADIAL_ETA * (distances[:, None] - shiftR) ** 2
    radial_terms = 0.25 * jnp.exp(-x2) * switch[:, None]
    radial_aev = jax.ops.segment_sum(
        radial_terms[:, :, None] * valence[edge_dst, None, :], edge_src,
        n_atoms).reshape(n_atoms, -1)

    d12 = 0.5 * (ang_distances[angle_src] + ang_distances[angle_dst])[:, None]
    angle_start = np.pi / (2 * ANGLE_SECTIONS)
    shiftZ = jnp.asarray(
        (np.linspace(0, np.pi, ANGLE_SECTIONS + 1) + angle_start)[None, :-1],
        dtype=ang_distances.dtype)
    shiftA = jnp.asarray(
        np.linspace(ANGULAR_START, ANGULAR_CUTOFF, ANGULAR_DIST_DIVISIONS + 1)[None, :-1],
        dtype=ang_distances.dtype)
    factor1 = (0.5 + 0.5 * jnp.cos(angles[:, None] - shiftZ)) ** ZETA
    factor2 = jnp.exp(-ANGULAR_ETA * (d12 - shiftA) ** 2)
    angular_terms = (factor1[:, None, :] * factor2[:, :, None]).reshape(
        -1, ANGLE_SECTIONS * ANGULAR_DIST_DIVISIONS) * 2 * (
        ang_switch[angle_src] * ang_switch[angle_dst])[:, None]
    valence_dst = valence[ang_edge_dst]
    valence_ang_p = valence_dst[angle_src] + valence_dst[angle_dst]
    valence_ang_m = valence_dst[angle_src] * valence_dst[angle_dst]
    valence_ang = (valence_ang_p[:, :, None] * valence_ang_m[:, None, :]).reshape(
        -1, valence_ang_p.shape[1] * valence_ang_m.shape[1])
    angular_aev = jax.ops.segment_sum(
        angular_terms[:, :, None] * valence_ang[:, None, :], central_atom,
        n_atoms).reshape(n_atoms, -1)

    onehot = jax.nn.one_hot(species, N_SPECIES, dtype=distances.dtype)
    return jnp.concatenate((onehot, radial_aev, angular_aev), axis=-1)


# speedup vs baseline: 1.5177x; 1.5177x over previous
import functools
import math
import numpy as np
import jax
import jax.numpy as jnp
from jax import lax
from jax.experimental import pallas as pl
from jax.experimental.pallas import tpu as pltpu
from jax.experimental.pallas import tpu_sc as plsc

N_NODES = 50000
N_EDGES = 800000
N_ANGLES = 200000
N_SPECIES = 10

RADIAL_ETA = 16.0
ANGULAR_ETA = 8.0
ZETA = 32.0
ANGLE_SECTIONS = 4
RADIAL_START = 0.8
ANGULAR_START = 0.8
CUTOFF = 5.2
ANGULAR_CUTOFF = 3.5

NC = 2
NS = 16
NW = NC * NS
CH = 128

N_EDGE_PAD = 802816
N_ANG_PAD = 212992
NROUND_R = N_EDGE_PAD // (NW * CH)
NROUND_A = N_ANG_PAD // (NW * CH)
NCH_PRE = N_ANG_PAD // CH

STRIPE = 3200
N_NODES_PAD = NS * STRIPE

_VAL = np.array([
    [0.0, 0.0, 0.0, 0.0],
    [0.5, 0.0, 0.0, 0.0],
    [1.0, 0.0, 0.0, 0.0],
    [0.5, 0.0, 0.0, 0.0],
    [1.0, 0.0, 0.0, 0.0],
    [1.0, 1.0 / 6.0, 0.0, 0.0],
    [1.0, 2.0 / 6.0, 0.0, 0.0],
    [1.0, 3.0 / 6.0, 0.0, 0.0],
    [1.0, 4.0 / 6.0, 0.0, 0.0],
    [1.0, 5.0 / 6.0, 0.0, 0.0],
], dtype=np.float64)

_VTAB_R = (0.25 * _VAL).astype(np.float32).reshape(-1)

_p = _VAL[:, None, :] + _VAL[None, :, :]
_m = _VAL[:, None, :] * _VAL[None, :, :]
_VTAB16 = (_p[:, :, :, None] * _m[:, :, None, :]).reshape(100, 16)
_VTAB16 = _VTAB16.astype(np.float32).reshape(-1)

_SHIFT_R = np.linspace(RADIAL_START, CUTOFF, 17)[:-1]
_SHIFT_Z = (np.linspace(0, np.pi, ANGLE_SECTIONS + 1)
            + np.pi / (2 * ANGLE_SECTIONS))[:-1]
_SHIFT_A = np.linspace(ANGULAR_START, ANGULAR_CUTOFF, 5)[:-1]

_COS_CO = [1.0] + [0.5 * (-1.0) ** k / float(math.factorial(2 * k))
                   for k in range(1, 9)]

_params = dict(
    compiler_params=pltpu.CompilerParams(needs_layout_passes=False,
                                         use_tc_tiling_on_sc=False),
)


@functools.cache
def _mesh():
    return plsc.VectorSubcoreMesh(core_axis_name="c", subcore_axis_name="s",
                                  num_cores=NC, num_subcores=NS)


def _iota16():
    return lax.iota(jnp.int32, 16)


def _prepass_body(angd_h, angsw_h, angedst_h, species_h, asrc_h, adst_h,
                  rec_d_h, rec_sw_h, rec_pv_h,
                  src_b, dst_b, d1_b, d2_b, w1_b, w2_b, e1_b, e2_b,
                  s1_b, s2_b, od_b, ow_b, op_b):
    c = lax.axis_index("c")
    s = lax.axis_index("s")
    wid = s * NC + c

    def chunk(j, carry):
        base = (wid + j * NW) * CH
        pltpu.sync_copy(asrc_h.at[pl.ds(base, CH)], src_b)
        pltpu.sync_copy(adst_h.at[pl.ds(base, CH)], dst_b)
        pltpu.sync_copy(angd_h.at[src_b], d1_b)
        pltpu.sync_copy(angd_h.at[dst_b], d2_b)
        pltpu.sync_copy(angsw_h.at[src_b], w1_b)
        pltpu.sync_copy(angsw_h.at[dst_b], w2_b)
        pltpu.sync_copy(angedst_h.at[src_b], e1_b)
        pltpu.sync_copy(angedst_h.at[dst_b], e2_b)
        pltpu.sync_copy(species_h.at[e1_b], s1_b)
        pltpu.sync_copy(species_h.at[e2_b], s2_b)
        for g in range(CH // 16):
            sl = pl.ds(g * 16, 16)
            od_b[sl] = 0.5 * (d1_b[sl] + d2_b[sl])
            gidx = base + (g * 16 + _iota16())
            live = gidx < N_ANGLES
            ow_b[sl] = jnp.where(live, 2.0 * (w1_b[sl] * w2_b[sl]), 0.0)
            op_b[sl] = (s1_b[sl] * 10 + s2_b[sl]) * 16
        pltpu.sync_copy(od_b, rec_d_h.at[pl.ds(base, CH)])
        pltpu.sync_copy(ow_b, rec_sw_h.at[pl.ds(base, CH)])
        pltpu.sync_copy(op_b, rec_pv_h.at[pl.ds(base, CH)])
        return carry

    lax.fori_loop(0, NCH_PRE // NW, chunk, jnp.int32(0))


@functools.cache
def _prepass():
    return pl.kernel(
        _prepass_body,
        out_type=(
            jax.ShapeDtypeStruct((N_ANG_PAD,), jnp.float32),
            jax.ShapeDtypeStruct((N_ANG_PAD,), jnp.float32),
            jax.ShapeDtypeStruct((N_ANG_PAD,), jnp.int32),
        ),
        mesh=_mesh(),
        scratch_types=[pltpu.VMEM((CH,), jnp.int32)] * 2
        + [pltpu.VMEM((CH,), jnp.float32)] * 4
        + [pltpu.VMEM((CH,), jnp.int32)] * 4
        + [pltpu.VMEM((CH,), jnp.float32)] * 2
        + [pltpu.VMEM((CH,), jnp.int32)],
        **_params,
    )


def _radial_body(dist_h, sw_h, src_h, dst_h, species_h, vtab_h, zeros_h,
                 out_h, acc, vtab_t, d_b, w_b, dst_b, sv_b, src_b, upd):
    c = lax.axis_index("c")
    s = lax.axis_index("s")
    wid = s * NC + c
    cf = c.astype(jnp.float32)

    pltpu.sync_copy(vtab_h, vtab_t)
    r0 = s * STRIPE
    pltpu.sync_copy(zeros_h, upd)
    for k in range(STRIPE // CH):
        pltpu.sync_copy(upd, acc.at[pl.ds(r0 + k * CH, CH)])
    plsc.subcore_barrier()

    rstep = float(_SHIFT_R[1] - _SHIFT_R[0])
    iota = _iota16()

    def rnd(j, carry):
        base = j * CH
        pltpu.sync_copy(dist_h.at[pl.ds(base, CH)], d_b)
        pltpu.sync_copy(sw_h.at[pl.ds(base, CH)], w_b)
        pltpu.sync_copy(dst_h.at[pl.ds(base, CH)], dst_b)
        pltpu.sync_copy(src_h.at[pl.ds(base, CH)], src_b)
        pltpu.sync_copy(species_h.at[dst_b], sv_b)
        for g in range(CH // 16):
            sl = pl.ds(g * 16, 16)
            dv = d_b[sl]
            wv = w_b[sl]
            voff = sv_b[sl] * 4
            vvals = [plsc.load_gather(vtab_t, [voff + cc]) for cc in range(4)]
            row_idx = iota + g * 16
            for r in range(8):
                sh = RADIAL_START + (cf * 8.0 + float(r)) * rstep
                x = dv - sh
                e = jnp.exp(x * x * (-RADIAL_ETA)) * wv
                for cc in range(4):
                    col = jnp.full((16,), r * 4 + cc, jnp.int32)
                    plsc.store_scatter(upd, [row_idx, col], e * vvals[cc])

        pltpu.sync_copy(upd, acc.at[src_b], add=True)
        return carry

    cnt_r = jnp.where(s == 0, jnp.int32(N_EDGE_PAD // CH), jnp.int32(0))
    lax.fori_loop(0, cnt_r, rnd, jnp.int32(0))
    plsc.subcore_barrier()
    pltpu.sync_copy(acc.at[pl.ds(r0, STRIPE)],
                    out_h.at[c, pl.ds(r0, STRIPE)])


@functools.cache
def _radial():
    return pl.kernel(
        _radial_body,
        out_type=jax.ShapeDtypeStruct((NC, N_NODES_PAD, 32), jnp.float32),
        mesh=_mesh(),
        scratch_types=[
            pltpu.VMEM_SHARED((N_NODES_PAD, 32), jnp.float32),
            pltpu.VMEM((40,), jnp.float32),
            pltpu.VMEM((CH,), jnp.float32),
            pltpu.VMEM((CH,), jnp.float32),
            pltpu.VMEM((CH,), jnp.int32),
            pltpu.VMEM((CH,), jnp.int32),
            pltpu.VMEM((CH,), jnp.int32),
            pltpu.VMEM((CH, 32), jnp.float32),
        ],
        **_params,
    )


def _cos_half(x):
    u = x * x
    h = jnp.full((16,), _COS_CO[8], jnp.float32)
    for k in range(7, -1, -1):
        h = h * u + _COS_CO[k]
    return h


def _angular_body(rec_d_h, rec_sw_h, rec_pv_h, angles_h, cen_h, vtab16_h,
                  zeros_h, out_h, acc, vtab_t,
                  d_b, w_b, a_b, pv_b, cen_b, upd):
    c = lax.axis_index("c")
    s = lax.axis_index("s")
    wid = s * NC + c
    cf = c.astype(jnp.float32)

    pltpu.sync_copy(vtab16_h, vtab_t)
    r0 = s * STRIPE
    astep = float(_SHIFT_A[1] - _SHIFT_A[0])
    iota = _iota16()

    for p in range(4):
        pltpu.sync_copy(zeros_h, upd)
        for k in range(STRIPE // CH):
            pltpu.sync_copy(upd, acc.at[pl.ds(r0 + k * CH, CH)])
        plsc.subcore_barrier()

        ia_l = p // 2
        sh_a = ANGULAR_START + (cf * 2.0 + float(ia_l)) * astep

        def rnd(j, carry):
            base = j * CH

            pltpu.sync_copy(rec_d_h.at[pl.ds(base, CH)], d_b)
            pltpu.sync_copy(rec_sw_h.at[pl.ds(base, CH)], w_b)
            pltpu.sync_copy(rec_pv_h.at[pl.ds(base, CH)], pv_b)
            pltpu.sync_copy(angles_h.at[pl.ds(base, CH)], a_b)
            pltpu.sync_copy(cen_h.at[pl.ds(base, CH)], cen_b)
            for g in range(CH // 16):
                sl = pl.ds(g * 16, 16)
                x = d_b[sl] - sh_a
                f2 = jnp.exp(x * x * (-ANGULAR_ETA)) * w_b[sl]
                pvv = pv_b[sl]
                av = a_b[sl]
                row_idx = iota + g * 16
                for jz in range(2):
                    iz = 2 * (p % 2) + jz
                    h = _cos_half(av - float(_SHIFT_Z[iz]))
                    b = h * h
                    b = b * b
                    b = b * b
                    b = b * b
                    b = b * b
                    tj = f2 * b
                    for kk in range(16):
                        vk = plsc.load_gather(vtab_t, [pvv + kk])
                        col = jnp.full((16,), jz * 16 + kk, jnp.int32)
                        plsc.store_scatter(upd, [row_idx, col], tj * vk)

            pltpu.sync_copy(upd, acc.at[cen_b], add=True)
            return carry

        cnt_a = jnp.where(s == 0, jnp.int32(N_ANG_PAD // CH), jnp.int32(0))
        lax.fori_loop(0, cnt_a, rnd, jnp.int32(0))
        plsc.subcore_barrier()
        pltpu.sync_copy(acc.at[pl.ds(r0, STRIPE)],
                        out_h.at[c, p, pl.ds(r0, STRIPE)])
        plsc.subcore_barrier()


@functools.cache
def _angular():
    return pl.kernel(
        _angular_body,
        out_type=jax.ShapeDtypeStruct((NC, 4, N_NODES_PAD, 32), jnp.float32),
        mesh=_mesh(),
        scratch_types=[
            pltpu.VMEM_SHARED((N_NODES_PAD, 32), jnp.float32),
            pltpu.VMEM((1600,), jnp.float32),
            pltpu.VMEM((CH,), jnp.float32),
            pltpu.VMEM((CH,), jnp.float32),
            pltpu.VMEM((CH,), jnp.float32),
            pltpu.VMEM((CH,), jnp.int32),
            pltpu.VMEM((CH,), jnp.int32),
            pltpu.VMEM((CH, 32), jnp.float32),
        ],
        **_params,
    )


def kernel(species, distances, switch, edge_src, edge_dst, angles,
           ang_distances, ang_switch, ang_edge_dst, central_atom,
           angle_src, angle_dst):
    species = species.astype(jnp.int32)
    edge_src = edge_src.astype(jnp.int32)
    edge_dst = edge_dst.astype(jnp.int32)
    ang_edge_dst = ang_edge_dst.astype(jnp.int32)
    central_atom = central_atom.astype(jnp.int32)
    angle_src = angle_src.astype(jnp.int32)
    angle_dst = angle_dst.astype(jnp.int32)

    vtab_r = jnp.asarray(_VTAB_R)
    vtab16 = jnp.asarray(_VTAB16)
    zeros = jnp.zeros((CH, 32), jnp.float32)

    epad = N_EDGE_PAD - N_EDGES
    dist_p = jnp.pad(distances, (0, epad), constant_values=1.0)
    sw_p = jnp.pad(switch, (0, epad))
    esrc_p = jnp.pad(edge_src, (0, epad))
    edst_p = jnp.pad(edge_dst, (0, epad))

    apad = N_ANG_PAD - N_ANGLES
    asrc_p = jnp.pad(angle_src, (0, apad))
    adst_p = jnp.pad(angle_dst, (0, apad))
    angles_p = jnp.pad(angles, (0, apad))
    cen_p = jnp.pad(central_atom, (0, apad))

    rec_d, rec_sw, rec_pv = _prepass()(
        ang_distances, ang_switch, ang_edge_dst, species, asrc_p, adst_p)

    rad = _radial()(dist_p, sw_p, esrc_p, edst_p, species, vtab_r, zeros)
    ang = _angular()(rec_d, rec_sw, rec_pv, angles_p, cen_p, vtab16, zeros)

    radial_aev = rad[:, :N_NODES].transpose(1, 0, 2).reshape(N_NODES, 64)
    angular_aev = ang[:, :, :N_NODES].transpose(2, 0, 1, 3).reshape(
        N_NODES, 256)
    onehot = jax.nn.one_hot(species, N_SPECIES, dtype=jnp.float32)
    return jnp.concatenate((onehot, radial_aev, angular_aev), axis=-1)
